# baseline (speedup 1.0000x reference)
import functools

import jax
import jax.numpy as jnp
from jax import lax
from jax.experimental import pallas as pl
from jax.experimental.pallas import tpu as pltpu
from jax.experimental.pallas import tpu_sc as plsc

NUM_HEADS = 16
SEQ = 2048
TBL = 1023
TBL_FLAT = TBL * NUM_HEADS
EXT_PITCH = 4352
NUM_SHIFTS = 8
LANES = 16
PAL_H = 128
PAL_W = 3968
PAL_ROWS_PER_WORKER = PAL_H // 2
FIRE = 8
CHUNKS = PAL_ROWS_PER_WORKER // FIRE


def _sc_body(table_hbm, pal_hbm, tbl_v, ext_v, sem):
    head = lax.axis_index("s")
    half = lax.axis_index("c")

    pltpu.sync_copy(table_hbm, tbl_v.at[pl.ds(0, TBL_FLAT)])

    lane = lax.iota(jnp.int32, LANES)

    def build(it, _):
        base = it * LANES
        pos = base + lane
        for b in range(NUM_SHIFTS):
            r_idx = jnp.clip(pos + (b - 1536), 0, TBL - 1)
            vals = plsc.load_gather(tbl_v, [r_idx * NUM_HEADS + head])
            ext_v[pl.ds(b * EXT_PITCH + base, LANES)] = vals
        return 0

    lax.fori_loop(0, EXT_PITCH // LANES, build, 0)

    row_base = half * PAL_ROWS_PER_WORKER

    def fire(c):
        for j in range(FIRE):
            r = row_base + c * FIRE + j
            q = (PAL_H - 1) - r
            b = lax.rem(q, NUM_SHIFTS)
            src_off = pl.multiple_of(b * EXT_PITCH + (q - b), 8)
            pltpu.async_copy(
                ext_v.at[pl.ds(src_off, PAL_W)],
                pal_hbm.at[head * PAL_H + r],
                sem)

    def drain():
        for _ in range(FIRE):
            pltpu.make_async_copy(
                pal_hbm.at[0],
                ext_v.at[pl.ds(0, PAL_W)],
                sem).wait()

    fire(0)

    def chunk(c, _):
        fire(c)
        drain()
        return 0

    lax.fori_loop(1, CHUNKS, chunk, 0)
    drain()


TC_BLK = 2048


def _tc_body(pal_ref, out_ref):
    for t in range(TC_BLK // PAL_H):
        off = 1920 - PAL_H * t
        out_ref[0, pl.ds(PAL_H * t, PAL_H)] = pal_ref[:, off:off + SEQ]


@jax.jit
def _materialize(table_flat):
    sc = functools.partial(
        pl.kernel,
        out_type=jax.ShapeDtypeStruct((NUM_HEADS * PAL_H, PAL_W), jnp.float32),
        mesh=plsc.VectorSubcoreMesh(core_axis_name="c", subcore_axis_name="s"),
        scratch_types=[
            pltpu.VMEM((16384,), jnp.float32),
            pltpu.VMEM((NUM_SHIFTS * EXT_PITCH,), jnp.float32),
            pltpu.SemaphoreType.DMA,
        ],
        compiler_params=pltpu.CompilerParams(
            needs_layout_passes=False, use_tc_tiling_on_sc=False),
    )(_sc_body)
    pal = sc(table_flat)

    out = pl.pallas_call(
        _tc_body,
        out_shape=jax.ShapeDtypeStruct((NUM_HEADS, SEQ, SEQ), jnp.float32),
        grid=(NUM_HEADS,),
        in_specs=[pl.BlockSpec((PAL_H, PAL_W), lambda h: (h, 0))],
        out_specs=pl.BlockSpec((1, TC_BLK, SEQ), lambda h: (h, 0, 0)),
    )(pal)
    return out


def kernel(bias_table, seq_len):
    del seq_len
    return _materialize(bias_table.reshape(-1))


# speedup vs baseline: 1.1572x; 1.1572x over previous
import functools

import jax
import jax.numpy as jnp
from jax import lax
from jax.experimental import pallas as pl
from jax.experimental.pallas import tpu as pltpu
from jax.experimental.pallas import tpu_sc as plsc

NUM_HEADS = 16
SEQ = 2048
TBL = 1023
TBL_FLAT = TBL * NUM_HEADS
EXT_PITCH = 4352
NUM_SHIFTS = 8
LANES = 16
PAL_H = 64
PAL_W = 4032
ROWS_PER_WORKER = PAL_H // 2
FIRE = 8
CHUNKS = ROWS_PER_WORKER // FIRE


def _sc_body(table_hbm, pal_hbm, tbl_v, ext_v, sem):
    head = lax.axis_index("s")
    half = lax.axis_index("c")

    pltpu.sync_copy(table_hbm, tbl_v.at[pl.ds(0, TBL_FLAT)])

    lane = lax.iota(jnp.int32, LANES)

    def build(it, _):
        base = it * LANES
        pos = base + lane
        for b in range(NUM_SHIFTS):
            r_idx = jnp.clip(pos + (b - 1536), 0, TBL - 1)
            vals = plsc.load_gather(tbl_v, [r_idx * NUM_HEADS + head])
            ext_v[pl.ds(b * EXT_PITCH + base, LANES)] = vals
        return 0

    lax.fori_loop(0, EXT_PITCH // LANES, build, 0)

    row_base = half * ROWS_PER_WORKER

    def fire(c):
        for j in range(FIRE):
            r = row_base + c * FIRE + j
            q = (PAL_H - 1) - r
            b = lax.rem(q, NUM_SHIFTS)
            src_off = pl.multiple_of(b * EXT_PITCH + (q - b), 8)
            pltpu.async_copy(
                ext_v.at[pl.ds(src_off, PAL_W)],
                pal_hbm.at[head * PAL_H + r],
                sem)

    def drain():
        for _ in range(FIRE):
            pltpu.make_async_copy(
                pal_hbm.at[0],
                ext_v.at[pl.ds(0, PAL_W)],
                sem).wait()

    fire(0)

    def chunk(c, _):
        fire(c)
        drain()
        return 0

    lax.fori_loop(1, CHUNKS, chunk, 0)
    drain()


def _tc_body(pal_ref, out_ref):
    for g in range(SEQ // PAL_H):
        off = (SEQ - PAL_H) - PAL_H * g
        out_ref[0, pl.ds(PAL_H * g, PAL_H)] = pal_ref[:, off:off + SEQ]


@jax.jit
def _materialize(table_flat):
    sc = functools.partial(
        pl.kernel,
        out_type=jax.ShapeDtypeStruct((NUM_HEADS * PAL_H, PAL_W), jnp.float32),
        mesh=plsc.VectorSubcoreMesh(core_axis_name="c", subcore_axis_name="s"),
        scratch_types=[
            pltpu.VMEM((16384,), jnp.float32),
            pltpu.VMEM((NUM_SHIFTS * EXT_PITCH,), jnp.float32),
            pltpu.SemaphoreType.DMA,
        ],
        compiler_params=pltpu.CompilerParams(
            needs_layout_passes=False, use_tc_tiling_on_sc=False),
    )(_sc_body)
    pal = sc(table_flat)

    out = pl.pallas_call(
        _tc_body,
        out_shape=jax.ShapeDtypeStruct((NUM_HEADS, SEQ, SEQ), jnp.float32),
        grid=(NUM_HEADS,),
        in_specs=[pl.BlockSpec((PAL_H, PAL_W), lambda h: (h, 0))],
        out_specs=pl.BlockSpec((1, SEQ, SEQ), lambda h: (h, 0, 0)),
    )(pal)
    return out


def kernel(bias_table, seq_len):
    del seq_len
    return _materialize(bias_table.reshape(-1))


# speedup vs baseline: 1.2615x; 1.0901x over previous
import functools

import jax
import jax.numpy as jnp
from jax import lax
from jax.experimental import pallas as pl
from jax.experimental.pallas import tpu as pltpu
from jax.experimental.pallas import tpu_sc as plsc

NUM_HEADS = 16
SEQ = 2048
TBL = 1023
TBL_FLAT = TBL * NUM_HEADS
EXT_PITCH = 4352
NUM_SHIFTS = 8
LANES = 16
PAL_H = 32
PAL_W = 4064
ROWS_PER_WORKER = PAL_H // 2
FIRE = 8
CHUNKS = ROWS_PER_WORKER // FIRE


def _sc_body(table_hbm, pal_hbm, tbl_v, ext_v, sem):
    head = lax.axis_index("s")
    half = lax.axis_index("c")

    pltpu.sync_copy(table_hbm, tbl_v.at[pl.ds(0, TBL_FLAT)])

    lane = lax.iota(jnp.int32, LANES)

    def build(it, _):
        base = it * LANES
        pos = base + lane
        for b in range(NUM_SHIFTS):
            r_idx = jnp.clip(pos + (b - 1536), 0, TBL - 1)
            vals = plsc.load_gather(tbl_v, [r_idx * NUM_HEADS + head])
            ext_v[pl.ds(b * EXT_PITCH + base, LANES)] = vals
        return 0

    lax.fori_loop(0, EXT_PITCH // LANES, build, 0)

    row_base = half * ROWS_PER_WORKER

    def fire(c):
        for j in range(FIRE):
            r = row_base + c * FIRE + j
            q = (PAL_H - 1) - r
            b = lax.rem(q, NUM_SHIFTS)
            src_off = pl.multiple_of(b * EXT_PITCH + (q - b), 8)
            pltpu.async_copy(
                ext_v.at[pl.ds(src_off, PAL_W)],
                pal_hbm.at[head * PAL_H + r],
                sem)

    def drain():
        for _ in range(FIRE):
            pltpu.make_async_copy(
                pal_hbm.at[0],
                ext_v.at[pl.ds(0, PAL_W)],
                sem).wait()

    fire(0)

    def chunk(c, _):
        fire(c)
        drain()
        return 0

    lax.fori_loop(1, CHUNKS, chunk, 0)
    drain()


def _tc_body(pal_ref, out_ref):
    for g in range(SEQ // PAL_H):
        off = (SEQ - PAL_H) - PAL_H * g
        out_ref[0, pl.ds(PAL_H * g, PAL_H)] = pal_ref[:, off:off + SEQ]


@jax.jit
def _materialize(table_flat):
    sc = functools.partial(
        pl.kernel,
        out_type=jax.ShapeDtypeStruct((NUM_HEADS * PAL_H, PAL_W), jnp.float32),
        mesh=plsc.VectorSubcoreMesh(core_axis_name="c", subcore_axis_name="s"),
        scratch_types=[
            pltpu.VMEM((16384,), jnp.float32),
            pltpu.VMEM((NUM_SHIFTS * EXT_PITCH,), jnp.float32),
            pltpu.SemaphoreType.DMA,
        ],
        compiler_params=pltpu.CompilerParams(
            needs_layout_passes=False, use_tc_tiling_on_sc=False),
    )(_sc_body)
    pal = sc(table_flat)

    out = pl.pallas_call(
        _tc_body,
        out_shape=jax.ShapeDtypeStruct((NUM_HEADS, SEQ, SEQ), jnp.float32),
        grid=(NUM_HEADS,),
        in_specs=[pl.BlockSpec((PAL_H, PAL_W), lambda h: (h, 0))],
        out_specs=pl.BlockSpec((1, SEQ, SEQ), lambda h: (h, 0, 0)),
    )(pal)
    return out


def kernel(bias_table, seq_len):
    del seq_len
    return _materialize(bias_table.reshape(-1))


# speedup vs baseline: 1.3169x; 1.0439x over previous
import functools

import jax
import jax.numpy as jnp
from jax import lax
from jax.experimental import pallas as pl
from jax.experimental.pallas import tpu as pltpu
from jax.experimental.pallas import tpu_sc as plsc

NUM_HEADS = 16
SEQ = 2048
TBL = 1023
TBL_FLAT = TBL * NUM_HEADS
EXT_PITCH = 4352
NUM_SHIFTS = 8
LANES = 16
PAL_H = 16
PAL_W = 4080
ROWS_PER_WORKER = PAL_H // 2
FIRE = 8
CHUNKS = ROWS_PER_WORKER // FIRE


def _sc_body(table_hbm, pal_hbm, tbl_v, ext_v, sem):
    head = lax.axis_index("s")
    half = lax.axis_index("c")

    pltpu.sync_copy(table_hbm, tbl_v.at[pl.ds(0, TBL_FLAT)])

    lane = lax.iota(jnp.int32, LANES)

    def build(it, _):
        base = it * LANES
        pos = base + lane
        for b in range(NUM_SHIFTS):
            r_idx = jnp.clip(pos + (b - 1536), 0, TBL - 1)
            vals = plsc.load_gather(tbl_v, [r_idx * NUM_HEADS + head])
            ext_v[pl.ds(b * EXT_PITCH + base, LANES)] = vals
        return 0

    lax.fori_loop(0, EXT_PITCH // LANES, build, 0)

    row_base = half * ROWS_PER_WORKER

    def fire(c):
        for j in range(FIRE):
            r = row_base + c * FIRE + j
            q = (PAL_H - 1) - r
            b = lax.rem(q, NUM_SHIFTS)
            src_off = pl.multiple_of(b * EXT_PITCH + (q - b), 8)
            pltpu.async_copy(
                ext_v.at[pl.ds(src_off, PAL_W)],
                pal_hbm.at[head * PAL_H + r],
                sem)

    def drain():
        for _ in range(FIRE):
            pltpu.make_async_copy(
                pal_hbm.at[0],
                ext_v.at[pl.ds(0, PAL_W)],
                sem).wait()

    fire(0)

    def chunk(c, _):
        fire(c)
        drain()
        return 0

    lax.fori_loop(1, CHUNKS, chunk, 0)
    drain()


def _tc_body(pal_ref, out_ref):
    for g in range(SEQ // PAL_H):
        off = (SEQ - PAL_H) - PAL_H * g
        out_ref[0, pl.ds(PAL_H * g, PAL_H)] = pal_ref[:, off:off + SEQ]


@jax.jit
def _materialize(table_flat):
    sc = functools.partial(
        pl.kernel,
        out_type=jax.ShapeDtypeStruct((NUM_HEADS * PAL_H, PAL_W), jnp.float32),
        mesh=plsc.VectorSubcoreMesh(core_axis_name="c", subcore_axis_name="s"),
        scratch_types=[
            pltpu.VMEM((16384,), jnp.float32),
            pltpu.VMEM((NUM_SHIFTS * EXT_PITCH,), jnp.float32),
            pltpu.SemaphoreType.DMA,
        ],
        compiler_params=pltpu.CompilerParams(
            needs_layout_passes=False, use_tc_tiling_on_sc=False),
    )(_sc_body)
    pal = sc(table_flat)

    out = pl.pallas_call(
        _tc_body,
        out_shape=jax.ShapeDtypeStruct((NUM_HEADS, SEQ, SEQ), jnp.float32),
        grid=(NUM_HEADS,),
        in_specs=[pl.BlockSpec((PAL_H, PAL_W), lambda h: (h, 0))],
        out_specs=pl.BlockSpec((1, SEQ, SEQ), lambda h: (h, 0, 0)),
    )(pal)
    return out


def kernel(bias_table, seq_len):
    del seq_len
    return _materialize(bias_table.reshape(-1))


# speedup vs baseline: 1.3308x; 1.0106x over previous
import functools

import jax
import jax.numpy as jnp
from jax import lax
from jax.experimental import pallas as pl
from jax.experimental.pallas import tpu as pltpu
from jax.experimental.pallas import tpu_sc as plsc

NUM_HEADS = 16
SEQ = 2048
TBL = 1023
TBL_FLAT = TBL * NUM_HEADS
EXT_PITCH = 4352
NUM_SHIFTS = 8
LANES = 16
PAL_H = 8
PAL_W = 4088
ROWS_PER_WORKER = PAL_H // 2
FIRE = 4
CHUNKS = ROWS_PER_WORKER // FIRE


def _sc_body(table_hbm, pal_hbm, tbl_v, ext_v, sem):
    head = lax.axis_index("s")
    half = lax.axis_index("c")

    pltpu.sync_copy(table_hbm, tbl_v.at[pl.ds(0, TBL_FLAT)])

    lane = lax.iota(jnp.int32, LANES)

    def build(it, _):
        base = it * LANES
        pos = base + lane
        for b in range(NUM_SHIFTS):
            r_idx = jnp.clip(pos + (b - 1536), 0, TBL - 1)
            vals = plsc.load_gather(tbl_v, [r_idx * NUM_HEADS + head])
            ext_v[pl.ds(b * EXT_PITCH + base, LANES)] = vals
        return 0

    lax.fori_loop(0, EXT_PITCH // LANES, build, 0)

    row_base = half * ROWS_PER_WORKER

    def fire(c):
        for j in range(FIRE):
            r = row_base + c * FIRE + j
            q = (PAL_H - 1) - r
            b = lax.rem(q, NUM_SHIFTS)
            src_off = pl.multiple_of(b * EXT_PITCH + (q - b), 8)
            pltpu.async_copy(
                ext_v.at[pl.ds(src_off, PAL_W)],
                pal_hbm.at[head * PAL_H + r],
                sem)

    def drain():
        for _ in range(FIRE):
            pltpu.make_async_copy(
                pal_hbm.at[0],
                ext_v.at[pl.ds(0, PAL_W)],
                sem).wait()

    fire(0)

    def chunk(c, _):
        fire(c)
        drain()
        return 0

    lax.fori_loop(1, CHUNKS, chunk, 0)
    drain()


def _tc_body(pal_ref, out_ref):
    for g in range(SEQ // PAL_H):
        off = (SEQ - PAL_H) - PAL_H * g
        out_ref[0, pl.ds(PAL_H * g, PAL_H)] = pal_ref[:, off:off + SEQ]


@jax.jit
def _materialize(table_flat):
    sc = functools.partial(
        pl.kernel,
        out_type=jax.ShapeDtypeStruct((NUM_HEADS * PAL_H, PAL_W), jnp.float32),
        mesh=plsc.VectorSubcoreMesh(core_axis_name="c", subcore_axis_name="s"),
        scratch_types=[
            pltpu.VMEM((16384,), jnp.float32),
            pltpu.VMEM((NUM_SHIFTS * EXT_PITCH,), jnp.float32),
            pltpu.SemaphoreType.DMA,
        ],
        compiler_params=pltpu.CompilerParams(
            needs_layout_passes=False, use_tc_tiling_on_sc=False),
    )(_sc_body)
    pal = sc(table_flat)

    out = pl.pallas_call(
        _tc_body,
        out_shape=jax.ShapeDtypeStruct((NUM_HEADS, SEQ, SEQ), jnp.float32),
        grid=(NUM_HEADS,),
        in_specs=[pl.BlockSpec((PAL_H, PAL_W), lambda h: (h, 0))],
        out_specs=pl.BlockSpec((1, SEQ, SEQ), lambda h: (h, 0, 0)),
    )(pal)
    return out


def kernel(bias_table, seq_len):
    del seq_len
    return _materialize(bias_table.reshape(-1))


# speedup vs baseline: 1.3907x; 1.0450x over previous
import functools

import jax
import jax.numpy as jnp
from jax import lax
from jax.experimental import pallas as pl
from jax.experimental.pallas import tpu as pltpu
from jax.experimental.pallas import tpu_sc as plsc

NUM_HEADS = 16
SEQ = 2048
TBL = 1023
TBL_FLAT = TBL * NUM_HEADS
EXT_PITCH = 4352
NUM_SHIFTS = 8
LANES = 16
PAL_H = 8
PAL_W = 4088
ROWS_PER_WORKER = PAL_H // 2
FIRE = 4
CHUNKS = ROWS_PER_WORKER // FIRE


def _sc_body(table_hbm, pal_hbm, tbl_v, ext_v, sem):
    head = lax.axis_index("s")
    half = lax.axis_index("c")

    pltpu.sync_copy(table_hbm, tbl_v.at[pl.ds(0, TBL_FLAT)])

    lane = lax.iota(jnp.int32, LANES)
    b_base = (1 - half) * (NUM_SHIFTS // 2)

    def build(it, _):
        base = it * LANES
        pos = base + lane
        for b_off in range(NUM_SHIFTS // 2):
            b = b_base + b_off
            r_idx = jnp.clip(pos + b + (-1536), 0, TBL - 1)
            vals = plsc.load_gather(tbl_v, [r_idx * NUM_HEADS + head])
            ext_v[pl.ds(b * EXT_PITCH + base, LANES)] = vals
        return 0

    lax.fori_loop(0, EXT_PITCH // LANES, build, 0)

    row_base = half * ROWS_PER_WORKER

    def fire(c):
        for j in range(FIRE):
            r = row_base + c * FIRE + j
            q = (PAL_H - 1) - r
            b = lax.rem(q, NUM_SHIFTS)
            src_off = pl.multiple_of(b * EXT_PITCH + (q - b), 8)
            pltpu.async_copy(
                ext_v.at[pl.ds(src_off, PAL_W)],
                pal_hbm.at[head * PAL_H + r],
                sem)

    def drain():
        for _ in range(FIRE):
            pltpu.make_async_copy(
                pal_hbm.at[0],
                ext_v.at[pl.ds(0, PAL_W)],
                sem).wait()

    fire(0)

    def chunk(c, _):
        fire(c)
        drain()
        return 0

    lax.fori_loop(1, CHUNKS, chunk, 0)
    drain()


def _tc_body(pal_ref, out_ref):
    for g in range(SEQ // PAL_H):
        off = (SEQ - PAL_H) - PAL_H * g
        out_ref[0, pl.ds(PAL_H * g, PAL_H)] = pal_ref[:, off:off + SEQ]


@jax.jit
def _materialize(table_flat):
    sc = functools.partial(
        pl.kernel,
        out_type=jax.ShapeDtypeStruct((NUM_HEADS * PAL_H, PAL_W), jnp.float32),
        mesh=plsc.VectorSubcoreMesh(core_axis_name="c", subcore_axis_name="s"),
        scratch_types=[
            pltpu.VMEM((16384,), jnp.float32),
            pltpu.VMEM((NUM_SHIFTS * EXT_PITCH,), jnp.float32),
            pltpu.SemaphoreType.DMA,
        ],
        compiler_params=pltpu.CompilerParams(
            needs_layout_passes=False, use_tc_tiling_on_sc=False),
    )(_sc_body)
    pal = sc(table_flat)

    out = pl.pallas_call(
        _tc_body,
        out_shape=jax.ShapeDtypeStruct((NUM_HEADS, SEQ, SEQ), jnp.float32),
        grid=(NUM_HEADS,),
        in_specs=[pl.BlockSpec((PAL_H, PAL_W), lambda h: (h, 0))],
        out_specs=pl.BlockSpec((1, SEQ, SEQ), lambda h: (h, 0, 0)),
    )(pal)
    return out


def kernel(bias_table, seq_len):
    del seq_len
    return _materialize(bias_table.reshape(-1))
